# Initial kernel scaffold; baseline (speedup 1.0000x reference)
import jax
import jax.numpy as jnp
from jax.experimental import pallas as pl


def kernel(x, emb_weight):
    raise NotImplementedError("write your pallas kernel here")


# speedup vs baseline: 1.5770x; 1.5770x over previous
import jax
import jax.numpy as jnp
from jax import lax
from jax.experimental import pallas as pl
from jax.experimental.pallas import tpu as pltpu

_NE = 1024
_D = 256
_B = 16
_T = 1024


def _vq_body(x_ref, emb_ref, embt_ref, st_ref, idx_ref, loss_ref, se_ref):
    b = pl.program_id(0)
    emb = emb_ref[...]

    @pl.when(b == 0)
    def _():
        se_ref[...] = jnp.sum(emb * emb, axis=1, keepdims=True)

    xb = x_ref[0]
    se = se_ref[...]
    sx = jnp.sum(xb * xb, axis=0, keepdims=True)

    mm = lax.dot_general(emb, xb, (((1,), (0,)), ((), ())),
                         preferred_element_type=jnp.float32)
    d = (sx + se) - 2.0 * mm

    dmin = jnp.min(d, axis=0, keepdims=True)
    rows = lax.broadcasted_iota(jnp.int32, (_NE, _T), 0)
    idxi = jnp.min(jnp.where(d == dmin, rows, _NE),
                   axis=0, keepdims=True)
    onehot = (rows == idxi).astype(jnp.float32)

    q = lax.dot_general(embt_ref[...], onehot, (((1,), (0,)), ((), ())),
                        preferred_element_type=jnp.float32)

    diff = q - xb
    st_ref[0] = xb + diff
    idx_ref[0] = idxi

    part = jnp.sum(diff * diff, keepdims=True).reshape(1, 1)

    @pl.when(b == 0)
    def _():
        loss_ref[...] = part

    @pl.when(b > 0)
    def _():
        loss_ref[...] = loss_ref[...] + part


def kernel(x, emb_weight):
    B, C, H, W = x.shape
    x3 = x.reshape(B, C, H * W)
    emb_t = emb_weight.T

    st, idx, losssum = pl.pallas_call(
        _vq_body,
        grid=(B,),
        in_specs=[
            pl.BlockSpec((1, C, H * W), lambda b: (b, 0, 0)),
            pl.BlockSpec((_NE, _D), lambda b: (0, 0)),
            pl.BlockSpec((_D, _NE), lambda b: (0, 0)),
        ],
        out_specs=[
            pl.BlockSpec((1, C, H * W), lambda b: (b, 0, 0)),
            pl.BlockSpec((1, 1, H * W), lambda b: (b, 0, 0)),
            pl.BlockSpec((1, 1), lambda b: (0, 0)),
        ],
        out_shape=[
            jax.ShapeDtypeStruct((B, C, H * W), jnp.float32),
            jax.ShapeDtypeStruct((B, 1, H * W), jnp.int32),
            jax.ShapeDtypeStruct((1, 1), jnp.float32),
        ],
        scratch_shapes=[pltpu.VMEM((_NE, 1), jnp.float32)],
    )(x3, emb_weight, emb_t)

    quantized_st = st.reshape(B, C, H, W)
    encoding_indices = idx.reshape(B, H, W)
    loss = losssum[0, 0] / jnp.float32(B * C * H * W)
    return quantized_st, loss, loss, encoding_indices
